# baseline (device time: 36437 ns/iter reference)
import os

import jax
import jax.numpy as jnp
from jax import lax
from jax.experimental import pallas as pl
from jax.experimental.pallas import tpu as pltpu

_COMPUTE_ONLY = bool(os.environ.get("KERNEL_COMPUTE_ONLY"))

N_DEV = 16
B, SQ, SKV, HQ, DH = 2, 128, 128, 64, 64
H_LOC = HQ // N_DEV
D_MODEL = 512
ROWS = B * SQ
CHUNK = ROWS // N_DEV


def kernel(x, Wq, K_ext, V_ext, Wo):
    def body(x_ref, wq_ref, k_hbm, v_hbm, wo_ref, out_ref,
             part_buf, rs_buf, gather_buf, k_vmem, v_vmem,
             send1, recv1, send2, recv2, kv_sems):
        my_pos = lax.axis_index("i")

        kv_copies = []
        for b in range(B):
            for h in range(H_LOC):
                h_idx = my_pos * H_LOC + h
                for i, (hbm, vmem) in enumerate(
                    ((k_hbm, k_vmem), (v_hbm, v_vmem))
                ):
                    cp = pltpu.make_async_copy(
                        hbm.at[b, :, h_idx, :],
                        vmem.at[b, h],
                        kv_sems.at[(b * H_LOC + h) * 2 + i],
                    )
                    cp.start()
                    kv_copies.append(cp)

        if not _COMPUTE_ONLY:
            barrier_sem = pltpu.get_barrier_semaphore()
            for k in range(1, N_DEV):
                pl.semaphore_signal(
                    barrier_sem, inc=1,
                    device_id=(jnp.remainder(my_pos + k, N_DEV),),
                    device_id_type=pl.DeviceIdType.MESH,
                )

        wq = wq_ref[:, :].astype(jnp.bfloat16)
        wo = wo_ref[:, :].astype(jnp.bfloat16)
        for b in range(B):
            xb = x_ref[b, :, :].astype(jnp.bfloat16)
            q_b = lax.dot_general(
                xb, wq, (((1,), (0,)), ((), ())),
                preferred_element_type=jnp.float32,
            )
            if b == 0:
                for cp in kv_copies:
                    cp.wait()
            ctx_h = []
            for h in range(H_LOC):
                q_bh = q_b[:, h * DH:(h + 1) * DH].astype(jnp.bfloat16)
                k_bh = k_vmem[b, h, :, :].astype(jnp.bfloat16)
                s = lax.dot_general(
                    q_bh, k_bh, (((1,), (1,)), ((), ())),
                    preferred_element_type=jnp.float32,
                ) * 0.125
                s = s - jnp.max(s, axis=1, keepdims=True)
                w = jnp.exp(s)
                w = (w / jnp.sum(w, axis=1, keepdims=True)).astype(jnp.bfloat16)
                v_bh = v_vmem[b, h, :, :].astype(jnp.bfloat16)
                ctx_h.append(lax.dot_general(
                    w, v_bh, (((1,), (0,)), ((), ())),
                    preferred_element_type=jnp.float32,
                ))
            ctx_b = jnp.concatenate(ctx_h, axis=1).astype(jnp.bfloat16)
            p_b = lax.dot_general(
                ctx_b, wo, (((1,), (0,)), ((), ())),
                preferred_element_type=jnp.float32,
            ).astype(jnp.bfloat16)
            part_buf[pl.ds(b * SQ, SQ), :] = p_b

            if _COMPUTE_ONLY:
                continue
            if b == 0:
                pl.semaphore_wait(barrier_sem, N_DEV - 1)
            lo, hi = b * (N_DEV // B), (b + 1) * (N_DEV // B)
            for k in range(1, N_DEV):
                tgt = jnp.remainder(my_pos + k, N_DEV)

                @pl.when(jnp.logical_and(tgt >= lo, tgt < hi))
                def _(k=k, tgt=tgt):
                    pltpu.make_async_remote_copy(
                        src_ref=part_buf.at[pl.ds(tgt * CHUNK, CHUNK), :],
                        dst_ref=rs_buf.at[pl.ds(my_pos * CHUNK, CHUNK), :],
                        send_sem=send1.at[k - 1],
                        recv_sem=recv1.at[k - 1],
                        device_id=(tgt,),
                        device_id_type=pl.DeviceIdType.MESH,
                    ).start()

        if _COMPUTE_ONLY:
            out_ref[:, :] = part_buf[:, :].astype(jnp.float32)
            return

        sends = []
        rs_buf[pl.ds(my_pos * CHUNK, CHUNK), :] = (
            part_buf[pl.ds(my_pos * CHUNK, CHUNK), :]
        )
        for k in range(1, N_DEV):
            src_dev = jnp.remainder(my_pos - k, N_DEV)
            pltpu.make_async_remote_copy(
                src_ref=part_buf.at[pl.ds(0, CHUNK), :],
                dst_ref=rs_buf.at[pl.ds(src_dev * CHUNK, CHUNK), :],
                send_sem=send1.at[k - 1],
                recv_sem=recv1.at[k - 1],
                device_id=(src_dev,),
                device_id_type=pl.DeviceIdType.MESH,
            ).wait_recv()

        acc = rs_buf[0:CHUNK, :].astype(jnp.float32)
        for s in range(1, N_DEV):
            acc = acc + rs_buf[s * CHUNK:(s + 1) * CHUNK, :].astype(jnp.float32)
        gather_buf[pl.ds(my_pos * CHUNK, CHUNK), :] = acc.astype(jnp.bfloat16)

        for k in range(1, N_DEV):
            tgt = jnp.remainder(my_pos + k, N_DEV)
            rdma = pltpu.make_async_remote_copy(
                src_ref=gather_buf.at[pl.ds(my_pos * CHUNK, CHUNK), :],
                dst_ref=gather_buf.at[pl.ds(my_pos * CHUNK, CHUNK), :],
                send_sem=send2.at[k - 1],
                recv_sem=recv2.at[k - 1],
                device_id=(tgt,),
                device_id_type=pl.DeviceIdType.MESH,
            )
            rdma.start()
            sends.append(rdma)
        out_ref[pl.ds(my_pos * CHUNK, CHUNK), :] = acc
        for k in range(1, N_DEV):
            src_dev = jnp.remainder(my_pos - k, N_DEV)
            pltpu.make_async_remote_copy(
                src_ref=gather_buf.at[pl.ds(0, CHUNK), :],
                dst_ref=gather_buf.at[pl.ds(src_dev * CHUNK, CHUNK), :],
                send_sem=send2.at[k - 1],
                recv_sem=recv2.at[k - 1],
                device_id=(src_dev,),
                device_id_type=pl.DeviceIdType.MESH,
            ).wait_recv()
            out_ref[pl.ds(src_dev * CHUNK, CHUNK), :] = (
                gather_buf[pl.ds(src_dev * CHUNK, CHUNK), :].astype(jnp.float32)
            )

        for k in range(1, N_DEV):
            pltpu.make_async_remote_copy(
                src_ref=part_buf.at[pl.ds(0, CHUNK), :],
                dst_ref=rs_buf.at[pl.ds(0, CHUNK), :],
                send_sem=send1.at[k - 1],
                recv_sem=recv1.at[k - 1],
                device_id=(my_pos,),
                device_id_type=pl.DeviceIdType.MESH,
            ).wait_send()
        for rdma in sends:
            rdma.wait_send()

    out = pl.pallas_call(
        body,
        out_shape=jax.ShapeDtypeStruct((ROWS, D_MODEL), jnp.float32),
        in_specs=[
            pl.BlockSpec(memory_space=pltpu.VMEM),
            pl.BlockSpec(memory_space=pltpu.VMEM),
            pl.BlockSpec(memory_space=pl.ANY),
            pl.BlockSpec(memory_space=pl.ANY),
            pl.BlockSpec(memory_space=pltpu.VMEM),
        ],
        out_specs=pl.BlockSpec(memory_space=pltpu.VMEM),
        scratch_shapes=[
            pltpu.VMEM((ROWS, D_MODEL), jnp.bfloat16),
            pltpu.VMEM((ROWS, D_MODEL), jnp.bfloat16),
            pltpu.VMEM((ROWS, D_MODEL), jnp.bfloat16),
            pltpu.VMEM((B, H_LOC, SKV, DH), jnp.float32),
            pltpu.VMEM((B, H_LOC, SKV, DH), jnp.float32),
            pltpu.SemaphoreType.DMA((N_DEV - 1,)),
            pltpu.SemaphoreType.DMA((N_DEV - 1,)),
            pltpu.SemaphoreType.DMA((N_DEV - 1,)),
            pltpu.SemaphoreType.DMA((N_DEV - 1,)),
            pltpu.SemaphoreType.DMA((B * H_LOC * 2,)),
        ],
        compiler_params=pltpu.CompilerParams(
            collective_id=None if _COMPUTE_ONLY else 0
        ),
    )(x, Wq, K_ext, V_ext, Wo)
    return out.reshape(B, SQ, D_MODEL)


# device time: 20902 ns/iter; 1.7432x vs baseline; 1.7432x over previous
import os

import jax
import jax.numpy as jnp
from jax import lax
from jax.experimental import pallas as pl
from jax.experimental.pallas import tpu as pltpu

_COMPUTE_ONLY = bool(os.environ.get("KERNEL_COMPUTE_ONLY"))

N_DEV = 16
B, SQ, SKV, HQ, DH = 2, 128, 128, 64, 64
H_LOC = HQ // N_DEV
D_MODEL = 512
ROWS = B * SQ
CHUNK = ROWS // N_DEV


def kernel(x, Wq, K_ext, V_ext, Wo):
    my = lax.axis_index("i")
    K_loc = jnp.transpose(
        lax.dynamic_slice_in_dim(K_ext, my * H_LOC, H_LOC, axis=2), (0, 2, 1, 3)
    )
    V_loc = jnp.transpose(
        lax.dynamic_slice_in_dim(V_ext, my * H_LOC, H_LOC, axis=2), (0, 2, 1, 3)
    )

    def body(x_ref, wq_ref, k_ref, v_ref, wo_ref, out_ref,
             part_buf, rs_buf, gather_buf,
             send1, recv1, send2, recv2):
        my_pos = lax.axis_index("i")

        if not _COMPUTE_ONLY:
            barrier_sem = pltpu.get_barrier_semaphore()
            for k in range(1, N_DEV):
                pl.semaphore_signal(
                    barrier_sem, inc=1,
                    device_id=(jnp.remainder(my_pos + k, N_DEV),),
                    device_id_type=pl.DeviceIdType.MESH,
                )

        wq = wq_ref[:, :].astype(jnp.bfloat16)
        wo = wo_ref[:, :].astype(jnp.bfloat16)
        for b in range(B):
            xb = x_ref[b, :, :].astype(jnp.bfloat16)
            q_b = lax.dot_general(
                xb, wq, (((1,), (0,)), ((), ())),
                preferred_element_type=jnp.float32,
            )
            ctx_h = []
            for h in range(H_LOC):
                q_bh = q_b[:, h * DH:(h + 1) * DH].astype(jnp.bfloat16)
                k_bh = k_ref[b, h, :, :].astype(jnp.bfloat16)
                s = lax.dot_general(
                    q_bh, k_bh, (((1,), (1,)), ((), ())),
                    preferred_element_type=jnp.float32,
                ) * 0.125
                s = s - jnp.max(s, axis=1, keepdims=True)
                w = jnp.exp(s)
                w = (w / jnp.sum(w, axis=1, keepdims=True)).astype(jnp.bfloat16)
                v_bh = v_ref[b, h, :, :].astype(jnp.bfloat16)
                ctx_h.append(lax.dot_general(
                    w, v_bh, (((1,), (0,)), ((), ())),
                    preferred_element_type=jnp.float32,
                ))
            ctx_b = jnp.concatenate(ctx_h, axis=1).astype(jnp.bfloat16)
            p_b = lax.dot_general(
                ctx_b, wo, (((1,), (0,)), ((), ())),
                preferred_element_type=jnp.float32,
            ).astype(jnp.bfloat16)
            part_buf[pl.ds(b * SQ, SQ), :] = p_b

            if _COMPUTE_ONLY:
                continue
            if b == 0:
                pl.semaphore_wait(barrier_sem, N_DEV - 1)
            lo, hi = b * (N_DEV // B), (b + 1) * (N_DEV // B)
            for k in range(1, N_DEV):
                tgt = jnp.remainder(my_pos + k, N_DEV)

                @pl.when(jnp.logical_and(tgt >= lo, tgt < hi))
                def _(k=k, tgt=tgt):
                    pltpu.make_async_remote_copy(
                        src_ref=part_buf.at[pl.ds(tgt * CHUNK, CHUNK), :],
                        dst_ref=rs_buf.at[pl.ds(my_pos * CHUNK, CHUNK), :],
                        send_sem=send1.at[k - 1],
                        recv_sem=recv1.at[k - 1],
                        device_id=(tgt,),
                        device_id_type=pl.DeviceIdType.MESH,
                    ).start()

        if _COMPUTE_ONLY:
            out_ref[:, :] = part_buf[:, :].astype(jnp.float32)
            return

        sends = []
        rs_buf[pl.ds(my_pos * CHUNK, CHUNK), :] = (
            part_buf[pl.ds(my_pos * CHUNK, CHUNK), :]
        )
        for k in range(1, N_DEV):
            src_dev = jnp.remainder(my_pos - k, N_DEV)
            pltpu.make_async_remote_copy(
                src_ref=part_buf.at[pl.ds(0, CHUNK), :],
                dst_ref=rs_buf.at[pl.ds(src_dev * CHUNK, CHUNK), :],
                send_sem=send1.at[k - 1],
                recv_sem=recv1.at[k - 1],
                device_id=(src_dev,),
                device_id_type=pl.DeviceIdType.MESH,
            ).wait_recv()

        acc = rs_buf[0:CHUNK, :].astype(jnp.float32)
        for s in range(1, N_DEV):
            acc = acc + rs_buf[s * CHUNK:(s + 1) * CHUNK, :].astype(jnp.float32)
        gather_buf[pl.ds(my_pos * CHUNK, CHUNK), :] = acc.astype(jnp.bfloat16)

        for k in range(1, N_DEV):
            tgt = jnp.remainder(my_pos + k, N_DEV)
            rdma = pltpu.make_async_remote_copy(
                src_ref=gather_buf.at[pl.ds(my_pos * CHUNK, CHUNK), :],
                dst_ref=gather_buf.at[pl.ds(my_pos * CHUNK, CHUNK), :],
                send_sem=send2.at[k - 1],
                recv_sem=recv2.at[k - 1],
                device_id=(tgt,),
                device_id_type=pl.DeviceIdType.MESH,
            )
            rdma.start()
            sends.append(rdma)
        out_ref[pl.ds(my_pos * CHUNK, CHUNK), :] = acc
        for k in range(1, N_DEV):
            src_dev = jnp.remainder(my_pos - k, N_DEV)
            pltpu.make_async_remote_copy(
                src_ref=gather_buf.at[pl.ds(0, CHUNK), :],
                dst_ref=gather_buf.at[pl.ds(src_dev * CHUNK, CHUNK), :],
                send_sem=send2.at[k - 1],
                recv_sem=recv2.at[k - 1],
                device_id=(src_dev,),
                device_id_type=pl.DeviceIdType.MESH,
            ).wait_recv()
            out_ref[pl.ds(src_dev * CHUNK, CHUNK), :] = (
                gather_buf[pl.ds(src_dev * CHUNK, CHUNK), :].astype(jnp.float32)
            )

        for k in range(1, N_DEV):
            pltpu.make_async_remote_copy(
                src_ref=part_buf.at[pl.ds(0, CHUNK), :],
                dst_ref=rs_buf.at[pl.ds(0, CHUNK), :],
                send_sem=send1.at[k - 1],
                recv_sem=recv1.at[k - 1],
                device_id=(my_pos,),
                device_id_type=pl.DeviceIdType.MESH,
            ).wait_send()
        for rdma in sends:
            rdma.wait_send()

    out = pl.pallas_call(
        body,
        out_shape=jax.ShapeDtypeStruct((ROWS, D_MODEL), jnp.float32),
        in_specs=[pl.BlockSpec(memory_space=pltpu.VMEM)] * 5,
        out_specs=pl.BlockSpec(memory_space=pltpu.VMEM),
        scratch_shapes=[
            pltpu.VMEM((ROWS, D_MODEL), jnp.bfloat16),
            pltpu.VMEM((ROWS, D_MODEL), jnp.bfloat16),
            pltpu.VMEM((ROWS, D_MODEL), jnp.bfloat16),
            pltpu.SemaphoreType.DMA((N_DEV - 1,)),
            pltpu.SemaphoreType.DMA((N_DEV - 1,)),
            pltpu.SemaphoreType.DMA((N_DEV - 1,)),
            pltpu.SemaphoreType.DMA((N_DEV - 1,)),
        ],
        compiler_params=pltpu.CompilerParams(
            collective_id=None if _COMPUTE_ONLY else 0
        ),
    )(x, Wq, K_loc, V_loc, Wo)
    return out.reshape(B, SQ, D_MODEL)
